# baseline (speedup 1.0000x reference)
import functools

import jax
import jax.numpy as jnp
from jax import lax
from jax.experimental import pallas as pl
from jax.experimental.pallas import tpu as pltpu
from jax.experimental.pallas import tpu_sc as plsc

NUM_CLASSES = 64
FDIM = 128
NUM_CORES = 2
NUM_SUBCORES = 16
CHUNK = 80


def _sc_segsum_body(feat_hbm, labels_hbm, zeros2d_hbm, zeros1d_hbm,
                    sums_out, counts_out, rows_v, idx_v, ones_v, zrow_v,
                    acc_sp, cnt_sp):
    c = lax.axis_index("c")
    s = lax.axis_index("s")
    n = feat_hbm.shape[0]
    per_core = n // NUM_CORES
    per_tile = per_core // NUM_SUBCORES
    nchunks = per_tile // CHUNK
    base = c * per_core + s * per_tile

    one = jnp.ones((16,), jnp.float32)
    for j in range(CHUNK // 16):
        ones_v[pl.ds(j * 16, 16)] = one

    @pl.when(s == 0)
    def _():
        pltpu.sync_copy(zeros2d_hbm, rows_v.at[pl.ds(0, NUM_CLASSES), :])
        pltpu.sync_copy(rows_v.at[pl.ds(0, NUM_CLASSES), :], acc_sp)
        pltpu.sync_copy(zeros1d_hbm, zrow_v)
        pltpu.sync_copy(zrow_v, cnt_sp)

    plsc.subcore_barrier()

    def chunk_body(g, _):
        row0 = base + g * CHUNK
        pltpu.sync_copy(feat_hbm.at[pl.ds(row0, CHUNK), :], rows_v)
        pltpu.sync_copy(labels_hbm.at[pl.ds(row0, CHUNK)], idx_v)
        pltpu.sync_copy(rows_v, acc_sp.at[idx_v], add=True)
        pltpu.sync_copy(ones_v, cnt_sp.at[idx_v], add=True)
        return _

    lax.fori_loop(0, nchunks, chunk_body, None)

    plsc.subcore_barrier()

    @pl.when(s == 0)
    def _():
        pltpu.sync_copy(acc_sp, sums_out.at[c])
        pltpu.sync_copy(cnt_sp, counts_out.at[c])


def _segment_sums_sc(support_features, support_labels):
    mesh = plsc.VectorSubcoreMesh(core_axis_name="c", subcore_axis_name="s")
    zeros2d = jnp.zeros((NUM_CLASSES, FDIM), jnp.float32)
    zeros1d = jnp.zeros((NUM_CLASSES,), jnp.float32)
    f = pl.kernel(
        _sc_segsum_body,
        out_type=[
            jax.ShapeDtypeStruct((NUM_CORES, NUM_CLASSES, FDIM), jnp.float32),
            jax.ShapeDtypeStruct((NUM_CORES, NUM_CLASSES), jnp.float32),
        ],
        mesh=mesh,
        scratch_types=[
            pltpu.VMEM((CHUNK, FDIM), jnp.float32),
            pltpu.VMEM((CHUNK,), jnp.int32),
            pltpu.VMEM((CHUNK,), jnp.float32),
            pltpu.VMEM((NUM_CLASSES,), jnp.float32),
            pltpu.VMEM_SHARED((NUM_CLASSES, FDIM), jnp.float32),
            pltpu.VMEM_SHARED((NUM_CLASSES,), jnp.float32),
        ],
    )
    return f(support_features, support_labels, zeros2d, zeros1d)


def _distance_body(sums_ref, counts_ref, q_ref, out_ref):
    sums = sums_ref[0] + sums_ref[1]
    cnt = counts_ref[0, :] + counts_ref[1, :]
    present = (cnt > 0.0).astype(jnp.float32)
    tri = (
        lax.broadcasted_iota(jnp.int32, (NUM_CLASSES, NUM_CLASSES), 0)
        < lax.broadcasted_iota(jnp.int32, (NUM_CLASSES, NUM_CLASSES), 1)
    ).astype(jnp.float32)
    rank = lax.dot_general(
        present[None, :], tri, (((1,), (0,)), ((), ())),
        preferred_element_type=jnp.float32,
    )
    row_iota = lax.broadcasted_iota(
        jnp.int32, (NUM_CLASSES, NUM_CLASSES), 0).astype(jnp.float32)
    perm = (
        jnp.where((row_iota == rank) & (present[None, :] > 0.0), 1.0, 0.0)
        / jnp.maximum(cnt, 1.0)[None, :]
    )
    protos = jnp.dot(perm, sums, preferred_element_type=jnp.float32)

    q = q_ref[...]
    qsq = jnp.sum(q * q, axis=1, keepdims=True)
    psq = jnp.sum(protos * protos, axis=1)
    cross = lax.dot_general(
        q, protos, (((1,), (1,)), ((), ())),
        preferred_element_type=jnp.float32,
    )
    dist = jnp.maximum(qsq + psq[None, :] - 2.0 * cross, 0.0)
    logits = -dist
    m = jnp.max(logits, axis=1, keepdims=True)
    shifted = logits - m
    lse = jnp.log(jnp.sum(jnp.exp(shifted), axis=1, keepdims=True))
    out_ref[...] = shifted - lse


def _distances(sums, counts, query_features, block_rows):
    nq = query_features.shape[0]
    assert nq % block_rows == 0
    nblocks = nq // block_rows
    return pl.pallas_call(
        _distance_body,
        grid=(nblocks,),
        in_specs=[
            pl.BlockSpec((NUM_CORES, NUM_CLASSES, FDIM), lambda i: (0, 0, 0)),
            pl.BlockSpec((NUM_CORES, NUM_CLASSES), lambda i: (0, 0)),
            pl.BlockSpec((block_rows, FDIM), lambda i: (i, 0)),
        ],
        out_specs=pl.BlockSpec((block_rows, NUM_CLASSES), lambda i: (i, 0)),
        out_shape=jax.ShapeDtypeStruct((nq, NUM_CLASSES), jnp.float32),
    )(sums, counts, query_features)


@functools.partial(jax.jit, static_argnames=())
def kernel(support_features, support_labels, query_features):
    sums, counts = _segment_sums_sc(support_features, support_labels)
    return _distances(sums, counts, query_features, 2000)


# speedup vs baseline: 1.3468x; 1.3468x over previous
import functools

import jax
import jax.numpy as jnp
from jax import lax
from jax.experimental import pallas as pl
from jax.experimental.pallas import tpu as pltpu
from jax.experimental.pallas import tpu_sc as plsc

NUM_CLASSES = 64
FDIM = 128
NUM_CORES = 2
NUM_SUBCORES = 16
CHUNK = 80


def _sc_segsum_body(feat_hbm, labels_hbm, zeros2d_hbm, zeros1d_hbm,
                    sums_out, counts_out, rows_v, idx_v, ones_v, zrow_v,
                    acc_sp, cnt_sp, lsem, ssem):
    c = lax.axis_index("c")
    s = lax.axis_index("s")
    n = feat_hbm.shape[0]
    per_core = n // NUM_CORES
    per_tile = per_core // NUM_SUBCORES
    nchunks = per_tile // CHUNK
    base = c * per_core + s * per_tile

    one = jnp.ones((16,), jnp.float32)
    for j in range(CHUNK // 16):
        ones_v[pl.ds(j * 16, 16)] = one

    @pl.when(s == 0)
    def _():
        pltpu.sync_copy(zeros2d_hbm, rows_v.at[0, pl.ds(0, NUM_CLASSES), :])
        pltpu.sync_copy(rows_v.at[0, pl.ds(0, NUM_CLASSES), :], acc_sp)
        pltpu.sync_copy(zeros1d_hbm, zrow_v)
        pltpu.sync_copy(zrow_v, cnt_sp)

    plsc.subcore_barrier()

    def start_load(g, b):
        row0 = base + g * CHUNK
        pltpu.async_copy(feat_hbm.at[pl.ds(row0, CHUNK), :],
                         rows_v.at[b], lsem)
        pltpu.async_copy(labels_hbm.at[pl.ds(row0, CHUNK)],
                         idx_v.at[b], lsem)

    start_load(0, 0)

    def chunk_body(g, carry):
        b = lax.rem(g, 2)
        pltpu.make_async_copy(feat_hbm.at[pl.ds(0, CHUNK), :],
                              rows_v.at[b], lsem).wait()
        pltpu.make_async_copy(labels_hbm.at[pl.ds(0, CHUNK)],
                              idx_v.at[b], lsem).wait()

        @pl.when(g < nchunks - 1)
        def _():
            start_load(g + 1, 1 - b)

        a1 = pltpu.async_copy(rows_v.at[b], acc_sp.at[idx_v.at[b]], ssem,
                              add=True)
        a2 = pltpu.async_copy(ones_v, cnt_sp.at[idx_v.at[b]], ssem,
                              add=True)
        a1.wait()
        a2.wait()
        return carry

    lax.fori_loop(0, nchunks, chunk_body, None)

    plsc.subcore_barrier()

    @pl.when(s == 0)
    def _():
        pltpu.sync_copy(acc_sp, sums_out.at[c])
        pltpu.sync_copy(cnt_sp, counts_out.at[c])


def _segment_sums_sc(support_features, support_labels):
    mesh = plsc.VectorSubcoreMesh(core_axis_name="c", subcore_axis_name="s")
    zeros2d = jnp.zeros((NUM_CLASSES, FDIM), jnp.float32)
    zeros1d = jnp.zeros((NUM_CLASSES,), jnp.float32)
    f = pl.kernel(
        _sc_segsum_body,
        out_type=[
            jax.ShapeDtypeStruct((NUM_CORES, NUM_CLASSES, FDIM), jnp.float32),
            jax.ShapeDtypeStruct((NUM_CORES, NUM_CLASSES), jnp.float32),
        ],
        mesh=mesh,
        scratch_types=[
            pltpu.VMEM((2, CHUNK, FDIM), jnp.float32),
            pltpu.VMEM((2, CHUNK), jnp.int32),
            pltpu.VMEM((CHUNK,), jnp.float32),
            pltpu.VMEM((NUM_CLASSES,), jnp.float32),
            pltpu.VMEM_SHARED((NUM_CLASSES, FDIM), jnp.float32),
            pltpu.VMEM_SHARED((NUM_CLASSES,), jnp.float32),
            pltpu.SemaphoreType.DMA,
            pltpu.SemaphoreType.DMA,
        ],
    )
    return f(support_features, support_labels, zeros2d, zeros1d)


def _distance_body(sums_ref, counts_ref, q_ref, out_ref):
    sums = sums_ref[0] + sums_ref[1]
    cnt = counts_ref[0, :] + counts_ref[1, :]
    present = (cnt > 0.0).astype(jnp.float32)
    tri = (
        lax.broadcasted_iota(jnp.int32, (NUM_CLASSES, NUM_CLASSES), 0)
        < lax.broadcasted_iota(jnp.int32, (NUM_CLASSES, NUM_CLASSES), 1)
    ).astype(jnp.float32)
    rank = lax.dot_general(
        present[None, :], tri, (((1,), (0,)), ((), ())),
        preferred_element_type=jnp.float32,
    )
    row_iota = lax.broadcasted_iota(
        jnp.int32, (NUM_CLASSES, NUM_CLASSES), 0).astype(jnp.float32)
    perm = (
        jnp.where((row_iota == rank) & (present[None, :] > 0.0), 1.0, 0.0)
        / jnp.maximum(cnt, 1.0)[None, :]
    )
    protos = jnp.dot(perm, sums, preferred_element_type=jnp.float32)

    q = q_ref[...]
    qsq = jnp.sum(q * q, axis=1, keepdims=True)
    psq = jnp.sum(protos * protos, axis=1)
    cross = lax.dot_general(
        q, protos, (((1,), (1,)), ((), ())),
        preferred_element_type=jnp.float32,
    )
    dist = jnp.maximum(qsq + psq[None, :] - 2.0 * cross, 0.0)
    logits = -dist
    m = jnp.max(logits, axis=1, keepdims=True)
    shifted = logits - m
    lse = jnp.log(jnp.sum(jnp.exp(shifted), axis=1, keepdims=True))
    out_ref[...] = shifted - lse


def _distances(sums, counts, query_features, block_rows):
    nq = query_features.shape[0]
    assert nq % block_rows == 0
    nblocks = nq // block_rows
    return pl.pallas_call(
        _distance_body,
        grid=(nblocks,),
        in_specs=[
            pl.BlockSpec((NUM_CORES, NUM_CLASSES, FDIM), lambda i: (0, 0, 0)),
            pl.BlockSpec((NUM_CORES, NUM_CLASSES), lambda i: (0, 0)),
            pl.BlockSpec((block_rows, FDIM), lambda i: (i, 0)),
        ],
        out_specs=pl.BlockSpec((block_rows, NUM_CLASSES), lambda i: (i, 0)),
        out_shape=jax.ShapeDtypeStruct((nq, NUM_CLASSES), jnp.float32),
    )(sums, counts, query_features)


@functools.partial(jax.jit, static_argnames=())
def kernel(support_features, support_labels, query_features):
    sums, counts = _segment_sums_sc(support_features, support_labels)
    return _distances(sums, counts, query_features, 2000)


# speedup vs baseline: 1.5811x; 1.1740x over previous
import functools

import jax
import jax.numpy as jnp
from jax import lax
from jax.experimental import pallas as pl
from jax.experimental.pallas import tpu as pltpu
from jax.experimental.pallas import tpu_sc as plsc

NUM_CLASSES = 64
FDIM = 128
NUM_CORES = 2
NUM_SUBCORES = 16
CHUNK = 80


def _sc_segsum_body(feat_hbm, labels_hbm, zeros2d_hbm, zeros1d_hbm,
                    sums_out, counts_out, rows_v, idx_v, ones_v, zrow_v,
                    acc_sp, cnt_sp, lsem, ssem):
    c = lax.axis_index("c")
    s = lax.axis_index("s")
    n = feat_hbm.shape[0]
    per_core = n // NUM_CORES
    per_tile = per_core // NUM_SUBCORES
    nchunks = per_tile // CHUNK
    base = c * per_core + s * per_tile

    one = jnp.ones((16,), jnp.float32)
    for j in range(CHUNK // 16):
        ones_v[pl.ds(j * 16, 16)] = one

    @pl.when(s == 0)
    def _():
        pltpu.sync_copy(zeros2d_hbm, rows_v.at[0, pl.ds(0, NUM_CLASSES), :])
        pltpu.sync_copy(rows_v.at[0, pl.ds(0, NUM_CLASSES), :], acc_sp)
        pltpu.sync_copy(zeros1d_hbm, zrow_v)
        pltpu.sync_copy(zrow_v, cnt_sp)

    plsc.subcore_barrier()

    def start_load(g, b):
        row0 = base + g * CHUNK
        pltpu.async_copy(feat_hbm.at[pl.ds(row0, CHUNK), :],
                         rows_v.at[b], lsem)
        pltpu.async_copy(labels_hbm.at[pl.ds(row0, CHUNK)],
                         idx_v.at[b], lsem)

    start_load(0, 0)

    def chunk_body(g, carry):
        b = lax.rem(g, 2)
        pltpu.make_async_copy(feat_hbm.at[pl.ds(0, CHUNK), :],
                              rows_v.at[b], lsem).wait()
        pltpu.make_async_copy(labels_hbm.at[pl.ds(0, CHUNK)],
                              idx_v.at[b], lsem).wait()

        @pl.when(g < nchunks - 1)
        def _():
            start_load(g + 1, 1 - b)

        a1 = pltpu.async_copy(rows_v.at[b], acc_sp.at[idx_v.at[b]], ssem,
                              add=True)
        a2 = pltpu.async_copy(ones_v, cnt_sp.at[idx_v.at[b]], ssem,
                              add=True)
        a1.wait()
        a2.wait()
        return carry

    lax.fori_loop(0, nchunks, chunk_body, None)

    plsc.subcore_barrier()

    @pl.when(s == 0)
    def _():
        pltpu.sync_copy(acc_sp, sums_out.at[c])
        pltpu.sync_copy(cnt_sp, counts_out.at[c])


def _segment_sums_sc(support_features, support_labels):
    mesh = plsc.VectorSubcoreMesh(core_axis_name="c", subcore_axis_name="s")
    zeros2d = jnp.zeros((NUM_CLASSES, FDIM), jnp.float32)
    zeros1d = jnp.zeros((NUM_CLASSES,), jnp.float32)
    f = pl.kernel(
        _sc_segsum_body,
        out_type=[
            jax.ShapeDtypeStruct((NUM_CORES, NUM_CLASSES, FDIM), jnp.float32),
            jax.ShapeDtypeStruct((NUM_CORES, NUM_CLASSES), jnp.float32),
        ],
        mesh=mesh,
        scratch_types=[
            pltpu.VMEM((2, CHUNK, FDIM), jnp.float32),
            pltpu.VMEM((2, CHUNK), jnp.int32),
            pltpu.VMEM((CHUNK,), jnp.float32),
            pltpu.VMEM((NUM_CLASSES,), jnp.float32),
            pltpu.VMEM_SHARED((NUM_CLASSES, FDIM), jnp.float32),
            pltpu.VMEM_SHARED((NUM_CLASSES,), jnp.float32),
            pltpu.SemaphoreType.DMA,
            pltpu.SemaphoreType.DMA,
        ],
    )
    return f(support_features, support_labels, zeros2d, zeros1d)


def _distance_body(sums_ref, counts_ref, q_ref, out_ref):
    sums = sums_ref[0] + sums_ref[1]
    cnt = counts_ref[0, :] + counts_ref[1, :]
    present = (cnt > 0.0).astype(jnp.float32)
    tri = (
        lax.broadcasted_iota(jnp.int32, (NUM_CLASSES, NUM_CLASSES), 0)
        < lax.broadcasted_iota(jnp.int32, (NUM_CLASSES, NUM_CLASSES), 1)
    ).astype(jnp.float32)
    rank = lax.dot_general(
        present[None, :], tri, (((1,), (0,)), ((), ())),
        preferred_element_type=jnp.float32,
    )
    row_iota = lax.broadcasted_iota(
        jnp.int32, (NUM_CLASSES, NUM_CLASSES), 0).astype(jnp.float32)
    perm = (
        jnp.where((row_iota == rank) & (present[None, :] > 0.0), 1.0, 0.0)
        / jnp.maximum(cnt, 1.0)[None, :]
    )
    protos = jnp.dot(perm, sums, preferred_element_type=jnp.float32)

    q = q_ref[...]
    psq = jnp.sum(protos * protos, axis=1)
    cross2 = lax.dot_general(
        q, protos + protos, (((1,), (1,)), ((), ())),
        preferred_element_type=jnp.float32,
    )
    neg_ones = jnp.full((FDIM, NUM_CLASSES), -1.0, jnp.float32)
    nqsq = jnp.dot(q * q, neg_ones, preferred_element_type=jnp.float32)
    dist = jnp.maximum(-nqsq + psq[None, :] - cross2, 0.0)
    logits = -dist
    m = jnp.max(logits, axis=1, keepdims=True)
    shifted = logits - m
    lse = jnp.log(jnp.sum(jnp.exp(shifted), axis=1, keepdims=True))
    out_ref[...] = shifted - lse


def _distances(sums, counts, query_features, block_rows):
    nq = query_features.shape[0]
    assert nq % block_rows == 0
    nblocks = nq // block_rows
    return pl.pallas_call(
        _distance_body,
        grid=(nblocks,),
        in_specs=[
            pl.BlockSpec((NUM_CORES, NUM_CLASSES, FDIM), lambda i: (0, 0, 0)),
            pl.BlockSpec((NUM_CORES, NUM_CLASSES), lambda i: (0, 0)),
            pl.BlockSpec((block_rows, FDIM), lambda i: (i, 0)),
        ],
        out_specs=pl.BlockSpec((block_rows, NUM_CLASSES), lambda i: (i, 0)),
        out_shape=jax.ShapeDtypeStruct((nq, NUM_CLASSES), jnp.float32),
    )(sums, counts, query_features)


@functools.partial(jax.jit, static_argnames=())
def kernel(support_features, support_labels, query_features):
    sums, counts = _segment_sums_sc(support_features, support_labels)
    return _distances(sums, counts, query_features, 16000)


# speedup vs baseline: 1.7515x; 1.1078x over previous
import functools

import jax
import jax.numpy as jnp
from jax import lax
from jax.experimental import pallas as pl
from jax.experimental.pallas import tpu as pltpu
from jax.experimental.pallas import tpu_sc as plsc

NUM_CLASSES = 64
FDIM = 128
NUM_CORES = 2
NUM_SUBCORES = 16
CHUNK = 128


def _sc_segsum_body(feat_hbm, labels_hbm, zeros2d_hbm, zeros1d_hbm,
                    sums_out, counts_out, rows_v, idx_v, ones_v, zrow_v,
                    idx_tail, acc_sp, cnt_sp, lsem, ssem):
    c = lax.axis_index("c")
    s = lax.axis_index("s")
    n = feat_hbm.shape[0]
    per_core = n // NUM_CORES
    per_tile = per_core // NUM_SUBCORES
    nchunks = per_tile // CHUNK
    base = c * per_core + s * per_tile

    one = jnp.ones((16,), jnp.float32)
    for j in range(CHUNK // 16):
        ones_v[pl.ds(j * 16, 16)] = one

    @pl.when(s == 0)
    def _():
        pltpu.sync_copy(zeros2d_hbm, rows_v.at[0, pl.ds(0, NUM_CLASSES), :])
        pltpu.sync_copy(rows_v.at[0, pl.ds(0, NUM_CLASSES), :], acc_sp)
        pltpu.sync_copy(zeros1d_hbm, zrow_v)
        pltpu.sync_copy(zrow_v, cnt_sp)

    plsc.subcore_barrier()

    def start_load(g, b):
        row0 = base + g * CHUNK
        pltpu.async_copy(feat_hbm.at[pl.ds(row0, CHUNK), :],
                         rows_v.at[b], lsem)
        pltpu.async_copy(labels_hbm.at[pl.ds(row0, CHUNK)],
                         idx_v.at[b], lsem)

    start_load(0, 0)

    def chunk_body(g, carry):
        b = lax.rem(g, 2)
        pltpu.make_async_copy(feat_hbm.at[pl.ds(0, CHUNK), :],
                              rows_v.at[b], lsem).wait()
        pltpu.make_async_copy(labels_hbm.at[pl.ds(0, CHUNK)],
                              idx_v.at[b], lsem).wait()

        @pl.when(g < nchunks - 1)
        def _():
            start_load(g + 1, 1 - b)

        a1 = pltpu.async_copy(rows_v.at[b], acc_sp.at[idx_v.at[b]], ssem,
                              add=True)
        a2 = pltpu.async_copy(ones_v, cnt_sp.at[idx_v.at[b]], ssem,
                              add=True)
        a1.wait()
        a2.wait()
        return carry

    lax.fori_loop(0, nchunks, chunk_body, None)

    tail = per_tile - nchunks * CHUNK
    if tail:
        row0 = base + nchunks * CHUNK
        pltpu.sync_copy(feat_hbm.at[pl.ds(row0, tail), :],
                        rows_v.at[0, pl.ds(0, tail), :])
        pltpu.sync_copy(labels_hbm.at[pl.ds(row0, tail)], idx_tail)
        pltpu.sync_copy(rows_v.at[0, pl.ds(0, tail), :],
                        acc_sp.at[idx_tail], add=True)
        pltpu.sync_copy(ones_v.at[pl.ds(0, tail)],
                        cnt_sp.at[idx_tail], add=True)

    plsc.subcore_barrier()

    @pl.when(s == 0)
    def _():
        pltpu.sync_copy(acc_sp, sums_out.at[c])
        pltpu.sync_copy(cnt_sp, counts_out.at[c])


def _segment_sums_sc(support_features, support_labels):
    mesh = plsc.VectorSubcoreMesh(core_axis_name="c", subcore_axis_name="s")
    zeros2d = jnp.zeros((NUM_CLASSES, FDIM), jnp.float32)
    zeros1d = jnp.zeros((NUM_CLASSES,), jnp.float32)
    f = pl.kernel(
        _sc_segsum_body,
        out_type=[
            jax.ShapeDtypeStruct((NUM_CORES, NUM_CLASSES, FDIM), jnp.float32),
            jax.ShapeDtypeStruct((NUM_CORES, NUM_CLASSES), jnp.float32),
        ],
        mesh=mesh,
        scratch_types=[
            pltpu.VMEM((2, CHUNK, FDIM), jnp.float32),
            pltpu.VMEM((2, CHUNK), jnp.int32),
            pltpu.VMEM((CHUNK,), jnp.float32),
            pltpu.VMEM((NUM_CLASSES,), jnp.float32),
            pltpu.VMEM((16,), jnp.int32),
            pltpu.VMEM_SHARED((NUM_CLASSES, FDIM), jnp.float32),
            pltpu.VMEM_SHARED((NUM_CLASSES,), jnp.float32),
            pltpu.SemaphoreType.DMA,
            pltpu.SemaphoreType.DMA,
        ],
    )
    return f(support_features, support_labels, zeros2d, zeros1d)


def _distance_body(sums_ref, counts_ref, q_ref, out_ref):
    sums = sums_ref[0] + sums_ref[1]
    cnt = counts_ref[0, :] + counts_ref[1, :]
    present = (cnt > 0.0).astype(jnp.float32)
    tri = (
        lax.broadcasted_iota(jnp.int32, (NUM_CLASSES, NUM_CLASSES), 0)
        < lax.broadcasted_iota(jnp.int32, (NUM_CLASSES, NUM_CLASSES), 1)
    ).astype(jnp.float32)
    rank = lax.dot_general(
        present[None, :], tri, (((1,), (0,)), ((), ())),
        preferred_element_type=jnp.float32,
    )
    row_iota = lax.broadcasted_iota(
        jnp.int32, (NUM_CLASSES, NUM_CLASSES), 0).astype(jnp.float32)
    perm = (
        jnp.where((row_iota == rank) & (present[None, :] > 0.0), 1.0, 0.0)
        / jnp.maximum(cnt, 1.0)[None, :]
    )
    protos = jnp.dot(perm, sums, preferred_element_type=jnp.float32)

    q = q_ref[...]
    psq = jnp.sum(protos * protos, axis=1)
    cross2 = lax.dot_general(
        q, protos + protos, (((1,), (1,)), ((), ())),
        preferred_element_type=jnp.float32,
    )
    neg_ones = jnp.full((FDIM, NUM_CLASSES), -1.0, jnp.float32)
    nqsq = jnp.dot(q * q, neg_ones, preferred_element_type=jnp.float32)
    dist = jnp.maximum(-nqsq + psq[None, :] - cross2, 0.0)
    logits = -dist
    m = jnp.max(logits, axis=1, keepdims=True)
    shifted = logits - m
    lse = jnp.log(jnp.sum(jnp.exp(shifted), axis=1, keepdims=True))
    out_ref[...] = shifted - lse


def _distances(sums, counts, query_features, block_rows):
    nq = query_features.shape[0]
    assert nq % block_rows == 0
    nblocks = nq // block_rows
    return pl.pallas_call(
        _distance_body,
        grid=(nblocks,),
        in_specs=[
            pl.BlockSpec((NUM_CORES, NUM_CLASSES, FDIM), lambda i: (0, 0, 0)),
            pl.BlockSpec((NUM_CORES, NUM_CLASSES), lambda i: (0, 0)),
            pl.BlockSpec((block_rows, FDIM), lambda i: (i, 0)),
        ],
        out_specs=pl.BlockSpec((block_rows, NUM_CLASSES), lambda i: (i, 0)),
        out_shape=jax.ShapeDtypeStruct((nq, NUM_CLASSES), jnp.float32),
    )(sums, counts, query_features)


@functools.partial(jax.jit, static_argnames=())
def kernel(support_features, support_labels, query_features):
    sums, counts = _segment_sums_sc(support_features, support_labels)
    return _distances(sums, counts, query_features, 16000)


# speedup vs baseline: 2.0040x; 1.1442x over previous
import functools

import jax
import jax.numpy as jnp
from jax import lax
from jax.experimental import pallas as pl
from jax.experimental.pallas import tpu as pltpu
from jax.experimental.pallas import tpu_sc as plsc

NUM_CLASSES = 64
FDIM = 128
NUM_CORES = 2
NUM_SUBCORES = 16
CHUNK = 128
SC_ROWS = 204800
TC_BLOCK = 2560


def _sc_segsum_body(feat_hbm, labels_hbm, zeros2d_hbm, zeros1d_hbm,
                    sums_out, counts_out, rows_v, idx_v, ones_v, zrow_v,
                    idx_tail, acc_sp, cnt_sp, lsem, ssem):
    c = lax.axis_index("c")
    s = lax.axis_index("s")
    per_core = SC_ROWS // NUM_CORES
    per_tile = per_core // NUM_SUBCORES
    nchunks = per_tile // CHUNK
    base = c * per_core + s * per_tile

    one = jnp.ones((16,), jnp.float32)
    for j in range(CHUNK // 16):
        ones_v[pl.ds(j * 16, 16)] = one

    @pl.when(s == 0)
    def _():
        pltpu.sync_copy(zeros2d_hbm, rows_v.at[0, pl.ds(0, NUM_CLASSES), :])
        pltpu.sync_copy(rows_v.at[0, pl.ds(0, NUM_CLASSES), :], acc_sp)
        pltpu.sync_copy(zeros1d_hbm, zrow_v)
        pltpu.sync_copy(zrow_v, cnt_sp)

    plsc.subcore_barrier()

    def start_load(g, b):
        row0 = base + g * CHUNK
        pltpu.async_copy(feat_hbm.at[pl.ds(row0, CHUNK), :],
                         rows_v.at[b], lsem)
        pltpu.async_copy(labels_hbm.at[pl.ds(row0, CHUNK)],
                         idx_v.at[b], lsem)

    start_load(0, 0)

    def chunk_body(g, carry):
        b = lax.rem(g, 2)
        pltpu.make_async_copy(feat_hbm.at[pl.ds(0, CHUNK), :],
                              rows_v.at[b], lsem).wait()
        pltpu.make_async_copy(labels_hbm.at[pl.ds(0, CHUNK)],
                              idx_v.at[b], lsem).wait()

        @pl.when(g < nchunks - 1)
        def _():
            start_load(g + 1, 1 - b)

        a1 = pltpu.async_copy(rows_v.at[b], acc_sp.at[idx_v.at[b]], ssem,
                              add=True)
        a2 = pltpu.async_copy(ones_v, cnt_sp.at[idx_v.at[b]], ssem,
                              add=True)
        a1.wait()
        a2.wait()
        return carry

    lax.fori_loop(0, nchunks, chunk_body, None)

    tail = per_tile - nchunks * CHUNK
    if tail:
        row0 = base + nchunks * CHUNK
        pltpu.sync_copy(feat_hbm.at[pl.ds(row0, tail), :],
                        rows_v.at[0, pl.ds(0, tail), :])
        pltpu.sync_copy(labels_hbm.at[pl.ds(row0, tail)], idx_tail)
        pltpu.sync_copy(rows_v.at[0, pl.ds(0, tail), :],
                        acc_sp.at[idx_tail], add=True)
        pltpu.sync_copy(ones_v.at[pl.ds(0, tail)],
                        cnt_sp.at[idx_tail], add=True)

    plsc.subcore_barrier()

    @pl.when(s == 0)
    def _():
        pltpu.sync_copy(acc_sp, sums_out.at[c])
        pltpu.sync_copy(cnt_sp, counts_out.at[c])


def _segment_sums_sc(support_features, support_labels):
    mesh = plsc.VectorSubcoreMesh(core_axis_name="c", subcore_axis_name="s")
    zeros2d = jnp.zeros((NUM_CLASSES, FDIM), jnp.float32)
    zeros1d = jnp.zeros((NUM_CLASSES,), jnp.float32)
    f = pl.kernel(
        _sc_segsum_body,
        out_type=[
            jax.ShapeDtypeStruct((NUM_CORES, NUM_CLASSES, FDIM), jnp.float32),
            jax.ShapeDtypeStruct((NUM_CORES, NUM_CLASSES), jnp.float32),
        ],
        mesh=mesh,
        scratch_types=[
            pltpu.VMEM((2, CHUNK, FDIM), jnp.float32),
            pltpu.VMEM((2, CHUNK), jnp.int32),
            pltpu.VMEM((CHUNK,), jnp.float32),
            pltpu.VMEM((NUM_CLASSES,), jnp.float32),
            pltpu.VMEM((16,), jnp.int32),
            pltpu.VMEM_SHARED((NUM_CLASSES, FDIM), jnp.float32),
            pltpu.VMEM_SHARED((NUM_CLASSES,), jnp.float32),
            pltpu.SemaphoreType.DMA,
            pltpu.SemaphoreType.DMA,
        ],
    )
    return f(support_features, support_labels, zeros2d, zeros1d)


def _tc_segsum_body(labels_ref, feat_ref, sums_ref, counts_ref):
    i = pl.program_id(0)
    labels = labels_ref[0, 0, :]
    feats = feat_ref[...]
    onehot = (
        lax.broadcasted_iota(jnp.int32, (NUM_CLASSES, TC_BLOCK), 0)
        == labels[None, :]
    ).astype(jnp.float32)
    partial = jnp.dot(onehot, feats, preferred_element_type=jnp.float32)
    ones_row = jnp.ones((1, TC_BLOCK), jnp.float32)
    cnt = lax.dot_general(
        ones_row, onehot, (((1,), (1,)), ((), ())),
        preferred_element_type=jnp.float32,
    )

    @pl.when(i == 0)
    def _():
        sums_ref[...] = jnp.zeros_like(sums_ref)
        counts_ref[...] = jnp.zeros_like(counts_ref)

    sums_ref[...] += partial
    counts_ref[...] += cnt


def _segment_sums_tc(support_features, support_labels):
    n = support_features.shape[0]
    ntc = n - SC_ROWS
    assert ntc % TC_BLOCK == 0
    nblocks = ntc // TC_BLOCK
    first = SC_ROWS // TC_BLOCK
    labels3d = support_labels.reshape(n // TC_BLOCK, 1, TC_BLOCK)
    return pl.pallas_call(
        _tc_segsum_body,
        grid=(nblocks,),
        in_specs=[
            pl.BlockSpec((1, 1, TC_BLOCK), lambda i: (i + first, 0, 0)),
            pl.BlockSpec((TC_BLOCK, FDIM), lambda i: (i + first, 0)),
        ],
        out_specs=[
            pl.BlockSpec((NUM_CLASSES, FDIM), lambda i: (0, 0)),
            pl.BlockSpec((1, NUM_CLASSES), lambda i: (0, 0)),
        ],
        out_shape=[
            jax.ShapeDtypeStruct((NUM_CLASSES, FDIM), jnp.float32),
            jax.ShapeDtypeStruct((1, NUM_CLASSES), jnp.float32),
        ],
    )(labels3d, support_features)


def _distance_body(sums_ref, counts_ref, tc_sums_ref, tc_counts_ref,
                   q_ref, out_ref):
    sums = sums_ref[0] + sums_ref[1] + tc_sums_ref[...]
    cnt = (counts_ref[0, :] + counts_ref[1, :]
           + tc_counts_ref[0, :])
    present = (cnt > 0.0).astype(jnp.float32)
    tri = (
        lax.broadcasted_iota(jnp.int32, (NUM_CLASSES, NUM_CLASSES), 0)
        < lax.broadcasted_iota(jnp.int32, (NUM_CLASSES, NUM_CLASSES), 1)
    ).astype(jnp.float32)
    rank = lax.dot_general(
        present[None, :], tri, (((1,), (0,)), ((), ())),
        preferred_element_type=jnp.float32,
    )
    row_iota = lax.broadcasted_iota(
        jnp.int32, (NUM_CLASSES, NUM_CLASSES), 0).astype(jnp.float32)
    perm = (
        jnp.where((row_iota == rank) & (present[None, :] > 0.0), 1.0, 0.0)
        / jnp.maximum(cnt, 1.0)[None, :]
    )
    protos = jnp.dot(perm, sums, preferred_element_type=jnp.float32)

    q = q_ref[...]
    psq = jnp.sum(protos * protos, axis=1)
    cross2 = lax.dot_general(
        q, protos + protos, (((1,), (1,)), ((), ())),
        preferred_element_type=jnp.float32,
    )
    neg_ones = jnp.full((FDIM, NUM_CLASSES), -1.0, jnp.float32)
    nqsq = jnp.dot(q * q, neg_ones, preferred_element_type=jnp.float32)
    dist = jnp.maximum(-nqsq + psq[None, :] - cross2, 0.0)
    logits = -dist
    m = jnp.max(logits, axis=1, keepdims=True)
    shifted = logits - m
    lse = jnp.log(jnp.sum(jnp.exp(shifted), axis=1, keepdims=True))
    out_ref[...] = shifted - lse


def _distances(sums, counts, tc_sums, tc_counts, query_features, block_rows):
    nq = query_features.shape[0]
    assert nq % block_rows == 0
    nblocks = nq // block_rows
    return pl.pallas_call(
        _distance_body,
        grid=(nblocks,),
        in_specs=[
            pl.BlockSpec((NUM_CORES, NUM_CLASSES, FDIM), lambda i: (0, 0, 0)),
            pl.BlockSpec((NUM_CORES, NUM_CLASSES), lambda i: (0, 0)),
            pl.BlockSpec((NUM_CLASSES, FDIM), lambda i: (0, 0)),
            pl.BlockSpec((1, NUM_CLASSES), lambda i: (0, 0)),
            pl.BlockSpec((block_rows, FDIM), lambda i: (i, 0)),
        ],
        out_specs=pl.BlockSpec((block_rows, NUM_CLASSES), lambda i: (i, 0)),
        out_shape=jax.ShapeDtypeStruct((nq, NUM_CLASSES), jnp.float32),
    )(sums, counts, tc_sums, tc_counts, query_features)


@functools.partial(jax.jit, static_argnames=())
def kernel(support_features, support_labels, query_features):
    sums, counts = _segment_sums_sc(support_features, support_labels)
    tc_sums, tc_counts = _segment_sums_tc(support_features, support_labels)
    return _distances(sums, counts, tc_sums, tc_counts, query_features, 16000)


# speedup vs baseline: 2.0534x; 1.0246x over previous
import functools

import jax
import jax.numpy as jnp
from jax import lax
from jax.experimental import pallas as pl
from jax.experimental.pallas import tpu as pltpu
from jax.experimental.pallas import tpu_sc as plsc

NUM_CLASSES = 64
FDIM = 128
NUM_CORES = 2
NUM_SUBCORES = 16
CHUNK = 128
SC_ROWS = 204800
TC_BLOCK = 2560


def _sc_segsum_body(feat_hbm, labels_hbm, zeros2d_hbm, zeros1d_hbm,
                    sums_out, counts_out, rows_v, idx_v, ones_v, zrow_v,
                    idx_tail, acc_sp, cnt_sp, lsem, ssem):
    c = lax.axis_index("c")
    s = lax.axis_index("s")
    per_core = SC_ROWS // NUM_CORES
    per_tile = per_core // NUM_SUBCORES
    nchunks = per_tile // CHUNK
    base = c * per_core + s * per_tile

    one = jnp.ones((16,), jnp.float32)
    for j in range(CHUNK // 16):
        ones_v[pl.ds(j * 16, 16)] = one

    @pl.when(s == 0)
    def _():
        pltpu.sync_copy(zeros2d_hbm, rows_v.at[0, pl.ds(0, NUM_CLASSES), :])
        pltpu.sync_copy(rows_v.at[0, pl.ds(0, NUM_CLASSES), :], acc_sp)
        pltpu.sync_copy(zeros1d_hbm, zrow_v)
        pltpu.sync_copy(zrow_v, cnt_sp)

    plsc.subcore_barrier()

    def start_load(g, b):
        row0 = base + g * CHUNK
        pltpu.async_copy(feat_hbm.at[pl.ds(row0, CHUNK), :],
                         rows_v.at[b], lsem)
        pltpu.async_copy(labels_hbm.at[pl.ds(row0, CHUNK)],
                         idx_v.at[b], lsem)

    start_load(0, 0)

    def wait_scatters(b2):
        pltpu.make_async_copy(rows_v.at[b2], acc_sp.at[idx_v.at[b2]],
                              ssem).wait()
        pltpu.make_async_copy(ones_v, cnt_sp.at[idx_v.at[b2]], ssem).wait()

    def chunk_body(g, carry):
        b = lax.rem(g, 3)
        pltpu.make_async_copy(feat_hbm.at[pl.ds(0, CHUNK), :],
                              rows_v.at[b], lsem).wait()
        pltpu.make_async_copy(labels_hbm.at[pl.ds(0, CHUNK)],
                              idx_v.at[b], lsem).wait()

        @pl.when(g >= 2)
        def _():
            wait_scatters(lax.rem(g + 1, 3))

        @pl.when(g < nchunks - 1)
        def _():
            start_load(g + 1, lax.rem(g + 1, 3))

        pltpu.async_copy(rows_v.at[b], acc_sp.at[idx_v.at[b]], ssem,
                         add=True)
        pltpu.async_copy(ones_v, cnt_sp.at[idx_v.at[b]], ssem, add=True)
        return carry

    lax.fori_loop(0, nchunks, chunk_body, None)
    wait_scatters(lax.rem(nchunks - 2, 3))
    wait_scatters(lax.rem(nchunks - 1, 3))

    tail = per_tile - nchunks * CHUNK
    if tail:
        row0 = base + nchunks * CHUNK
        pltpu.sync_copy(feat_hbm.at[pl.ds(row0, tail), :],
                        rows_v.at[0, pl.ds(0, tail), :])
        pltpu.sync_copy(labels_hbm.at[pl.ds(row0, tail)], idx_tail)
        pltpu.sync_copy(rows_v.at[0, pl.ds(0, tail), :],
                        acc_sp.at[idx_tail], add=True)
        pltpu.sync_copy(ones_v.at[pl.ds(0, tail)],
                        cnt_sp.at[idx_tail], add=True)

    plsc.subcore_barrier()

    @pl.when(s == 0)
    def _():
        pltpu.sync_copy(acc_sp, sums_out.at[c])
        pltpu.sync_copy(cnt_sp, counts_out.at[c])


def _segment_sums_sc(support_features, support_labels):
    mesh = plsc.VectorSubcoreMesh(core_axis_name="c", subcore_axis_name="s")
    zeros2d = jnp.zeros((NUM_CLASSES, FDIM), jnp.float32)
    zeros1d = jnp.zeros((NUM_CLASSES,), jnp.float32)
    f = pl.kernel(
        _sc_segsum_body,
        out_type=[
            jax.ShapeDtypeStruct((NUM_CORES, NUM_CLASSES, FDIM), jnp.float32),
            jax.ShapeDtypeStruct((NUM_CORES, NUM_CLASSES), jnp.float32),
        ],
        mesh=mesh,
        scratch_types=[
            pltpu.VMEM((3, CHUNK, FDIM), jnp.float32),
            pltpu.VMEM((3, CHUNK), jnp.int32),
            pltpu.VMEM((CHUNK,), jnp.float32),
            pltpu.VMEM((NUM_CLASSES,), jnp.float32),
            pltpu.VMEM((16,), jnp.int32),
            pltpu.VMEM_SHARED((NUM_CLASSES, FDIM), jnp.float32),
            pltpu.VMEM_SHARED((NUM_CLASSES,), jnp.float32),
            pltpu.SemaphoreType.DMA,
            pltpu.SemaphoreType.DMA,
        ],
    )
    return f(support_features, support_labels, zeros2d, zeros1d)


def _tc_segsum_body(labels_ref, feat_ref, sums_ref, counts_ref):
    i = pl.program_id(0)
    labels = labels_ref[0, 0, :]
    feats = feat_ref[...]
    onehot = (
        lax.broadcasted_iota(jnp.int32, (NUM_CLASSES, TC_BLOCK), 0)
        == labels[None, :]
    ).astype(jnp.float32)
    partial = jnp.dot(onehot, feats, preferred_element_type=jnp.float32)
    ones_row = jnp.ones((1, TC_BLOCK), jnp.float32)
    cnt = lax.dot_general(
        ones_row, onehot, (((1,), (1,)), ((), ())),
        preferred_element_type=jnp.float32,
    )

    @pl.when(i == 0)
    def _():
        sums_ref[...] = jnp.zeros_like(sums_ref)
        counts_ref[...] = jnp.zeros_like(counts_ref)

    sums_ref[...] += partial
    counts_ref[...] += cnt


def _segment_sums_tc(support_features, support_labels):
    n = support_features.shape[0]
    ntc = n - SC_ROWS
    assert ntc % TC_BLOCK == 0
    nblocks = ntc // TC_BLOCK
    first = SC_ROWS // TC_BLOCK
    labels3d = support_labels.reshape(n // TC_BLOCK, 1, TC_BLOCK)
    return pl.pallas_call(
        _tc_segsum_body,
        grid=(nblocks,),
        in_specs=[
            pl.BlockSpec((1, 1, TC_BLOCK), lambda i: (i + first, 0, 0)),
            pl.BlockSpec((TC_BLOCK, FDIM), lambda i: (i + first, 0)),
        ],
        out_specs=[
            pl.BlockSpec((NUM_CLASSES, FDIM), lambda i: (0, 0)),
            pl.BlockSpec((1, NUM_CLASSES), lambda i: (0, 0)),
        ],
        out_shape=[
            jax.ShapeDtypeStruct((NUM_CLASSES, FDIM), jnp.float32),
            jax.ShapeDtypeStruct((1, NUM_CLASSES), jnp.float32),
        ],
    )(labels3d, support_features)


def _distance_body(sums_ref, counts_ref, tc_sums_ref, tc_counts_ref,
                   q_ref, out_ref):
    sums = sums_ref[0] + sums_ref[1] + tc_sums_ref[...]
    cnt = (counts_ref[0, :] + counts_ref[1, :]
           + tc_counts_ref[0, :])
    present = (cnt > 0.0).astype(jnp.float32)
    tri = (
        lax.broadcasted_iota(jnp.int32, (NUM_CLASSES, NUM_CLASSES), 0)
        < lax.broadcasted_iota(jnp.int32, (NUM_CLASSES, NUM_CLASSES), 1)
    ).astype(jnp.float32)
    rank = lax.dot_general(
        present[None, :], tri, (((1,), (0,)), ((), ())),
        preferred_element_type=jnp.float32,
    )
    row_iota = lax.broadcasted_iota(
        jnp.int32, (NUM_CLASSES, NUM_CLASSES), 0).astype(jnp.float32)
    perm = (
        jnp.where((row_iota == rank) & (present[None, :] > 0.0), 1.0, 0.0)
        / jnp.maximum(cnt, 1.0)[None, :]
    )
    protos = jnp.dot(perm, sums, preferred_element_type=jnp.float32)

    q = q_ref[...]
    psq = jnp.sum(protos * protos, axis=1)
    cross2 = lax.dot_general(
        q, protos + protos, (((1,), (1,)), ((), ())),
        preferred_element_type=jnp.float32,
    )
    neg_ones = jnp.full((FDIM, NUM_CLASSES), -1.0, jnp.float32)
    nqsq = jnp.dot(q * q, neg_ones, preferred_element_type=jnp.float32)
    logits = jnp.minimum(cross2 + nqsq - psq[None, :], 0.0)
    m = jnp.max(logits, axis=1, keepdims=True)
    shifted = logits - m
    lse = jnp.log(jnp.sum(jnp.exp(shifted), axis=1, keepdims=True))
    out_ref[...] = shifted - lse


def _distances(sums, counts, tc_sums, tc_counts, query_features, block_rows):
    nq = query_features.shape[0]
    assert nq % block_rows == 0
    nblocks = nq // block_rows
    return pl.pallas_call(
        _distance_body,
        grid=(nblocks,),
        in_specs=[
            pl.BlockSpec((NUM_CORES, NUM_CLASSES, FDIM), lambda i: (0, 0, 0)),
            pl.BlockSpec((NUM_CORES, NUM_CLASSES), lambda i: (0, 0)),
            pl.BlockSpec((NUM_CLASSES, FDIM), lambda i: (0, 0)),
            pl.BlockSpec((1, NUM_CLASSES), lambda i: (0, 0)),
            pl.BlockSpec((block_rows, FDIM), lambda i: (i, 0)),
        ],
        out_specs=pl.BlockSpec((block_rows, NUM_CLASSES), lambda i: (i, 0)),
        out_shape=jax.ShapeDtypeStruct((nq, NUM_CLASSES), jnp.float32),
    )(sums, counts, tc_sums, tc_counts, query_features)


@functools.partial(jax.jit, static_argnames=())
def kernel(support_features, support_labels, query_features):
    sums, counts = _segment_sums_sc(support_features, support_labels)
    tc_sums, tc_counts = _segment_sums_tc(support_features, support_labels)
    return _distances(sums, counts, tc_sums, tc_counts, query_features, 16000)
